# baseline (speedup 1.0000x reference)
import functools

import jax
import jax.numpy as jnp
from jax import lax
from jax.experimental import pallas as pl
from jax.experimental.pallas import tpu as pltpu
from jax.experimental.pallas import tpu_sc as plsc

N = 10000
E = 320000
D = 128

NC = 2
NS = 16
NW = NC * NS
EPW = E // NW
CHUNK = 128
NPAD = 10240
HALF = 5120
ACCR = HALF + 128
RPT = ACCR // NS
OPT = HALF // NS
CAPCH = 82
CAP = CAPCH * CHUNK
LANES = 16
TRASHW = HALF << 16

_mesh = plsc.VectorSubcoreMesh(
    core_axis_name="c", subcore_axis_name="s", num_cores=NC, num_subcores=NS
)


def _part_body(src_hbm, dst_hbm, hist_out, lists_out, cnts_out,
               src_v, dst_v, hs_v, hd_v, lpa_v, lpb_v, cnt_v):
    c = lax.axis_index("c")
    s = lax.axis_index("s")
    w = c * NS + s
    pltpu.sync_copy(src_hbm.at[w], src_v)
    pltpu.sync_copy(dst_hbm.at[w], dst_v)

    zero = jnp.zeros((LANES,), jnp.float32)
    itrash = jnp.full((LANES,), TRASHW, jnp.int32)

    @pl.loop(0, NPAD // LANES)
    def _zero_hist(i):
        hs_v[pl.ds(i * LANES, LANES)] = zero
        hd_v[pl.ds(i * LANES, LANES)] = zero

    @pl.loop(0, CAP // LANES)
    def _fill_lists(i):
        sl = pl.ds(i * LANES, LANES)
        lpa_v[sl] = itrash
        lpb_v[sl] = itrash

    one = jnp.ones((LANES,), jnp.float32)
    half_vec = jnp.full((LANES,), HALF, jnp.int32)

    def _step(i, carry):
        ofa, ofb = carry
        sl = pl.ds(i * LANES, LANES)
        sv = src_v[sl]
        dv = dst_v[sl]
        plsc.addupdate_scatter(hs_v, [sv], one)
        plsc.addupdate_scatter(hd_v, [dv], one)
        ma = dv < HALF
        mb = jnp.logical_not(ma)
        pka = sv | lax.shift_left(dv, 16)
        pkb = sv | lax.shift_left(dv - half_vec, 16)
        plsc.store_compressed(lpa_v.at[pl.ds(ofa, LANES)], pka, mask=ma)
        plsc.store_compressed(lpb_v.at[pl.ds(ofb, LANES)], pkb, mask=mb)
        pca = plsc.all_reduce_population_count(ma)[0]
        return ofa + pca, ofb + (LANES - pca)

    zi = jnp.int32(0)
    loop = pl.loop(0, EPW // LANES, init_carry=(zi, zi))
    ofa, ofb = loop(_step)

    ncha = (ofa + (CHUNK - 1)) // CHUNK
    nchb = (ofb + (CHUNK - 1)) // CHUNK
    iv = lax.broadcasted_iota(jnp.int32, (LANES,), 0)
    cnt_vec = jnp.where(iv == 0, ncha, jnp.where(iv == 1, nchb, 0))
    cnt_v[pl.ds(0, LANES)] = cnt_vec

    pltpu.sync_copy(hs_v, hist_out.at[w, 0])
    pltpu.sync_copy(hd_v, hist_out.at[w, 1])
    pltpu.sync_copy(lpa_v, lists_out.at[w, 0])
    pltpu.sync_copy(lpb_v, lists_out.at[w, 1])
    pltpu.sync_copy(cnt_v, cnts_out.at[w])


_part_call = pl.kernel(
    _part_body,
    out_type=(
        jax.ShapeDtypeStruct((NW, 2, NPAD), jnp.float32),
        jax.ShapeDtypeStruct((NW, 2, CAP), jnp.int32),
        jax.ShapeDtypeStruct((NW, LANES), jnp.int32),
    ),
    mesh=_mesh,
    compiler_params=pltpu.CompilerParams(needs_layout_passes=False),
    scratch_types=[
        pltpu.VMEM((EPW,), jnp.int32),
        pltpu.VMEM((EPW,), jnp.int32),
        pltpu.VMEM((NPAD,), jnp.float32),
        pltpu.VMEM((NPAD,), jnp.float32),
        pltpu.VMEM((CAP,), jnp.int32),
        pltpu.VMEM((CAP,), jnp.int32),
        pltpu.VMEM((LANES,), jnp.int32),
    ],
)


def _scatter_body(msg_hbm, lists_hbm, cnts_hbm, parts_out,
                  pk_v, cnts_v, isrc_v, idst_v,
                  rows0_v, rows1_v, acc_sh, gsem0, gsem1):
    c = lax.axis_index("c")
    s = lax.axis_index("s")

    zero = jnp.zeros((LANES,), jnp.float32)

    @pl.loop(0, CHUNK)
    def _zrow(i):
        @pl.loop(0, D // LANES)
        def _zcol(k):
            rows0_v[i, pl.ds(k * LANES, LANES)] = zero

    for off, ln in ((0, 128), (128, 128), (256, RPT - 256)):
        pltpu.sync_copy(rows0_v.at[pl.ds(0, ln)],
                        acc_sh.at[pl.ds(s * RPT + off, ln)])

    mask16 = jnp.full((LANES,), 0xFFFF, jnp.int32)
    for t in range(2):
        p = 2 * s + t
        pltpu.sync_copy(lists_hbm.at[p, c], pk_v)
        pltpu.sync_copy(cnts_hbm.at[p], cnts_v.at[t])

        @pl.loop(0, CAPCH)
        def _unpack(j):
            for k in range(CHUNK // LANES):
                sl = pl.ds(k * LANES, LANES)
                wv = pk_v[pl.ds(j * CHUNK + k * LANES, LANES)]
                isrc_v[t, j, sl] = wv & mask16
                idst_v[t, j, sl] = lax.shift_right_logical(wv, 16)

    plsc.subcore_barrier()

    def _consume(t):
        cv = cnts_v[t, pl.ds(0, LANES)]
        nch = jnp.where(c == 0, cv[0], cv[1])
        npair = (nch + 1) // 2
        pltpu.async_copy(msg_hbm.at[isrc_v.at[t, 0]], rows0_v, gsem0)
        pltpu.async_copy(msg_hbm.at[isrc_v.at[t, 1]], rows1_v, gsem1)

        @pl.loop(0, npair)
        def _pair(u):
            j = u * 2
            pltpu.make_async_copy(msg_hbm.at[isrc_v.at[t, j]], rows0_v,
                                  gsem0).wait()
            pltpu.sync_copy(rows0_v, acc_sh.at[idst_v.at[t, j]], add=True)
            pltpu.async_copy(msg_hbm.at[isrc_v.at[t, j + 2]], rows0_v, gsem0)

            pltpu.make_async_copy(msg_hbm.at[isrc_v.at[t, j + 1]], rows1_v,
                                  gsem1).wait()
            pltpu.sync_copy(rows1_v, acc_sh.at[idst_v.at[t, j + 1]], add=True)
            pltpu.async_copy(msg_hbm.at[isrc_v.at[t, j + 3]], rows1_v, gsem1)

        j_end = npair * 2
        pltpu.make_async_copy(msg_hbm.at[isrc_v.at[t, j_end]], rows0_v,
                              gsem0).wait()
        pltpu.make_async_copy(msg_hbm.at[isrc_v.at[t, j_end + 1]], rows1_v,
                              gsem1).wait()

    _consume(0)
    _consume(1)

    plsc.subcore_barrier()
    for off, ln in ((0, 128), (128, 128), (256, OPT - 256)):
        rows = pl.ds(s * OPT + off, ln)
        pltpu.sync_copy(acc_sh.at[rows], rows0_v.at[pl.ds(0, ln)])
        pltpu.sync_copy(rows0_v.at[pl.ds(0, ln)], parts_out.at[c, rows])


_scatter_call = pl.kernel(
    _scatter_body,
    out_type=jax.ShapeDtypeStruct((NC, HALF, D), jnp.float32),
    mesh=_mesh,
    compiler_params=pltpu.CompilerParams(
        needs_layout_passes=False, use_tc_tiling_on_sc=False
    ),
    scratch_types=[
        pltpu.VMEM((CAP,), jnp.int32),
        pltpu.VMEM((2, LANES), jnp.int32),
        pltpu.VMEM((2, CAPCH, CHUNK), jnp.int32),
        pltpu.VMEM((2, CAPCH, CHUNK), jnp.int32),
        pltpu.VMEM((CHUNK, D), jnp.float32),
        pltpu.VMEM((CHUNK, D), jnp.float32),
        pltpu.VMEM_SHARED((ACCR, D), jnp.float32),
        pltpu.SemaphoreType.DMA,
        pltpu.SemaphoreType.DMA,
    ],
)


def _msg_body(x_ref, wn_ref, bn_ref, we_ref, be_ref, hist_ref, out_ref):
    out_deg = jnp.sum(hist_ref[:, 0, :N], axis=0)
    scale = lax.rsqrt(jnp.maximum(out_deg, 1.0))
    h = jnp.dot(x_ref[...], wn_ref[...], preferred_element_type=jnp.float32)
    h = h + bn_ref[...]
    m = jnp.dot(h, we_ref[...], preferred_element_type=jnp.float32)
    m = m + be_ref[...]
    out_ref[...] = m * scale[:, None]


_msg_call = pl.pallas_call(
    _msg_body,
    out_shape=jax.ShapeDtypeStruct((N, D), jnp.float32),
)


def _out_body(parts_ref, hist_ref, wo_ref, bo_ref, out_ref):
    in_deg = jnp.sum(hist_ref[:, 1, :N], axis=0)
    nrm = lax.rsqrt(jnp.maximum(in_deg, 1.0))
    upd = jnp.concatenate([parts_ref[0], parts_ref[1, :N - HALF]], axis=0)
    upd = upd * nrm[:, None]
    z = jnp.dot(upd, wo_ref[...], preferred_element_type=jnp.float32)
    z = z + bo_ref[...]
    out_ref[...] = z * 0.5 * (1.0 + lax.erf(z * (2.0 ** -0.5)))


_out_call = pl.pallas_call(
    _out_body,
    out_shape=jax.ShapeDtypeStruct((N, D), jnp.float32),
)


def kernel(x, edge_index, W_node, b_node, W_edge, b_edge, W_out, b_out):
    src = edge_index[0].astype(jnp.int32)
    dst = edge_index[1].astype(jnp.int32)

    hist, lists, cnts = _part_call(src.reshape(NW, EPW), dst.reshape(NW, EPW))

    msg = _msg_call(x, W_node, b_node.reshape(1, D), W_edge,
                    b_edge.reshape(1, D), hist)

    parts = _scatter_call(msg, lists, cnts)

    return _out_call(parts, hist, W_out, b_out.reshape(1, D))


# speedup vs baseline: 4.0391x; 4.0391x over previous
import functools

import jax
import jax.numpy as jnp
from jax import lax
from jax.experimental import pallas as pl
from jax.experimental.pallas import tpu as pltpu
from jax.experimental.pallas import tpu_sc as plsc

N = 10000
E = 320000
D = 128
DH = D // 2

NC = 2
NS = 16
NW = NC * NS
EPW = E // NW
EPS = E // NS
CHUNK = 125
NCHUNK = EPS // CHUNK
NPAD = 10240
RPW = NPAD // NS
RCH = 128
LANES = 16

_mesh = plsc.VectorSubcoreMesh(
    core_axis_name="c", subcore_axis_name="s", num_cores=NC, num_subcores=NS
)


def _deg_body(src_hbm, dst_hbm, hist_out, src_v, dst_v, hs_v, hd_v):
    c = lax.axis_index("c")
    s = lax.axis_index("s")
    w = c * NS + s
    pltpu.sync_copy(src_hbm.at[w], src_v)
    pltpu.sync_copy(dst_hbm.at[w], dst_v)
    zero = jnp.zeros((LANES,), jnp.float32)

    @pl.loop(0, NPAD // LANES)
    def _zero(i):
        hs_v[pl.ds(i * LANES, LANES)] = zero
        hd_v[pl.ds(i * LANES, LANES)] = zero

    one = jnp.ones((LANES,), jnp.float32)

    @pl.loop(0, EPW // LANES)
    def _accum(i):
        si = src_v[pl.ds(i * LANES, LANES)]
        di = dst_v[pl.ds(i * LANES, LANES)]
        plsc.addupdate_scatter(hs_v, [si], one)
        plsc.addupdate_scatter(hd_v, [di], one)

    pltpu.sync_copy(hs_v, hist_out.at[w, 0])
    pltpu.sync_copy(hd_v, hist_out.at[w, 1])


_deg_call = pl.kernel(
    _deg_body,
    out_type=jax.ShapeDtypeStruct((NW, 2, NPAD), jnp.float32),
    mesh=_mesh,
    compiler_params=pltpu.CompilerParams(needs_layout_passes=False),
    scratch_types=[
        pltpu.VMEM((EPW,), jnp.int32),
        pltpu.VMEM((EPW,), jnp.int32),
        pltpu.VMEM((NPAD,), jnp.float32),
        pltpu.VMEM((NPAD,), jnp.float32),
    ],
)


NBUF = 2


def _scatter_body(msg0_hbm, msg1_hbm, srcw_hbm, dstw_hbm, parts_out,
                  src_v, dst_v, rows0_v, rows1_v, rows2_v, rows3_v,
                  stage_v, acc_sh,
                  gsem0, gsem1, gsem2, gsem3, ssem0, ssem1, ssem2, ssem3):
    rows = (rows0_v, rows1_v, rows2_v, rows3_v)
    gsem = (gsem0, gsem1, gsem2, gsem3)
    ssem = (ssem0, ssem1, ssem2, ssem3)
    c = lax.axis_index("c")
    s = lax.axis_index("s")
    pltpu.sync_copy(srcw_hbm.at[s], src_v)
    pltpu.sync_copy(dstw_hbm.at[s], dst_v)

    zero = jnp.zeros((LANES,), jnp.float32)

    @pl.loop(0, RCH)
    def _zrow(i):
        @pl.loop(0, DH // LANES)
        def _zcol(k):
            stage_v[i, pl.ds(k * LANES, LANES)] = zero

    for k in range(RPW // RCH):
        pltpu.sync_copy(stage_v, acc_sh.at[pl.ds(s * RPW + k * RCH, RCH)])
    plsc.subcore_barrier()

    def _edge_loop(msg_hbm):
        for b in range(NBUF):
            pltpu.async_copy(msg_hbm.at[src_v.at[b]], rows[b], gsem[b])

        @pl.loop(0, NCHUNK // NBUF)
        def _edge_chunk(kk):
            j = kk * NBUF
            for b in range(NBUF):
                pltpu.make_async_copy(msg_hbm.at[src_v.at[j + b]], rows[b],
                                      gsem[b]).wait()
                pltpu.sync_copy(rows[b], acc_sh.at[dst_v.at[j + b]],
                                add=True)
                pltpu.async_copy(msg_hbm.at[src_v.at[j + NBUF + b]],
                                 rows[b], gsem[b])

        for b in range(NBUF):
            pltpu.make_async_copy(msg_hbm.at[src_v.at[NCHUNK + b]], rows[b],
                                  gsem[b]).wait()

    @pl.when(c == 0)
    def _core0():
        _edge_loop(msg0_hbm)

    @pl.when(c == 1)
    def _core1():
        _edge_loop(msg1_hbm)

    plsc.subcore_barrier()
    for k in range(RPW // RCH):
        pltpu.sync_copy(acc_sh.at[pl.ds(s * RPW + k * RCH, RCH)], stage_v)
        pltpu.sync_copy(stage_v, parts_out.at[c, pl.ds(s * RPW + k * RCH, RCH)])


_scatter_call = pl.kernel(
    _scatter_body,
    out_type=jax.ShapeDtypeStruct((NC, NPAD, DH), jnp.float32),
    mesh=_mesh,
    compiler_params=pltpu.CompilerParams(
        needs_layout_passes=False, use_tc_tiling_on_sc=False
    ),
    scratch_types=[
        pltpu.VMEM((NCHUNK + NBUF, CHUNK), jnp.int32),
        pltpu.VMEM((NCHUNK, CHUNK), jnp.int32),
        pltpu.VMEM((CHUNK, DH), jnp.float32),
        pltpu.VMEM((CHUNK, DH), jnp.float32),
        pltpu.VMEM((CHUNK, DH), jnp.float32),
        pltpu.VMEM((CHUNK, DH), jnp.float32),
        pltpu.VMEM((RCH, DH), jnp.float32),
        pltpu.VMEM_SHARED((NPAD, DH), jnp.float32),
    ] + [pltpu.SemaphoreType.DMA] * 8,
)


def _msg_body(x_ref, wn_ref, bn_ref, we_ref, be_ref, hist_ref,
              out0_ref, out1_ref):
    out_deg = jnp.sum(hist_ref[:, 0, :N], axis=0)
    scale = lax.rsqrt(jnp.maximum(out_deg, 1.0))
    h = jnp.dot(x_ref[...], wn_ref[...], preferred_element_type=jnp.float32)
    h = h + bn_ref[...]
    m = jnp.dot(h, we_ref[...], preferred_element_type=jnp.float32)
    m = m + be_ref[...]
    m = m * scale[:, None]
    out0_ref[...] = m[:, :DH]
    out1_ref[...] = m[:, DH:]


_msg_call = pl.pallas_call(
    _msg_body,
    out_shape=(
        jax.ShapeDtypeStruct((N, DH), jnp.float32),
        jax.ShapeDtypeStruct((N, DH), jnp.float32),
    ),
)


def _out_body(parts_ref, hist_ref, wo_ref, bo_ref, out_ref):
    in_deg = jnp.sum(hist_ref[:, 1, :N], axis=0)
    nrm = lax.rsqrt(jnp.maximum(in_deg, 1.0))
    upd = jnp.concatenate([parts_ref[0, :N, :], parts_ref[1, :N, :]], axis=1)
    upd = upd * nrm[:, None]
    z = jnp.dot(upd, wo_ref[...], preferred_element_type=jnp.float32)
    z = z + bo_ref[...]
    out_ref[...] = z * 0.5 * (1.0 + lax.erf(z * (2.0 ** -0.5)))


_out_call = pl.pallas_call(
    _out_body,
    out_shape=jax.ShapeDtypeStruct((N, D), jnp.float32),
)


def kernel(x, edge_index, W_node, b_node, W_edge, b_edge, W_out, b_out):
    src = edge_index[0].astype(jnp.int32)
    dst = edge_index[1].astype(jnp.int32)

    hist = _deg_call(src.reshape(NW, EPW), dst.reshape(NW, EPW))

    msg0, msg1 = _msg_call(x, W_node, b_node.reshape(1, D), W_edge,
                           b_edge.reshape(1, D), hist)

    srcw = jnp.pad(src.reshape(NS, NCHUNK, CHUNK),
                   ((0, 0), (0, NBUF), (0, 0)))
    dstw = dst.reshape(NS, NCHUNK, CHUNK)
    parts = _scatter_call(msg0, msg1, srcw, dstw)

    return _out_call(parts, hist, W_out, b_out.reshape(1, D))
